# baseline (speedup 1.0000x reference)
import functools

import jax
import jax.numpy as jnp
from jax import lax
from jax.experimental import pallas as pl
from jax.experimental.pallas import tpu as pltpu
from jax.experimental.pallas import tpu_sc as plsc

B, P, C = 16384, 26, 128
NC, NS, L = 2, 16, 16
NW = NC * NS
BATCH_PER_W = B // NW
NB = 8
RPC = NB * P
NBUF = 4
CHUNKS = BATCH_PER_W // NB
IDXCOL = BATCH_PER_W + L


def _body(idx_hbm, out_hbm, idx_v, rows0, rows1, rows2, rows3,
          sem_i, sem0, sem1, sem2, sem3):
    wid = lax.axis_index("s") * NC + lax.axis_index("c")
    base_b = wid * BATCH_PER_W

    def stage_idx(p):
        pltpu.async_copy(
            idx_hbm.at[p, pl.ds(base_b, BATCH_PER_W)],
            idx_v.at[p, pl.ds(0, BATCH_PER_W)], sem_i)

    pl.loop(0, P)(stage_idx)

    zeros = jnp.zeros((L,), jnp.float32)
    ones = jnp.ones((L,), jnp.float32)
    lanes = lax.iota(jnp.int32, L)
    lo_mask = lanes < NB

    bufs = (rows0, rows1, rows2, rows3)
    sems = (sem0, sem1, sem2, sem3)

    def zero_buf(buf):
        def zrow(i):
            for k in range(C // L):
                buf[i, pl.ds(k * L, L)] = zeros
        pl.loop(0, RPC)(zrow)

    def scatter(chunk, buf, val):
        def srow(p):
            iv = idx_v[p, pl.ds(chunk * NB, L)]
            plsc.store_scatter(buf, [lanes + p * NB, iv], val,
                               mask=lo_mask)
        pl.loop(0, P)(srow)

    def fire(chunk, buf, sem):
        def frow(p):
            dst = out_hbm.at[p, pl.ds(base_b + chunk * NB, NB)]
            pltpu.async_copy(buf.at[pl.ds(p * NB, NB)], dst, sem)
        pl.loop(0, P)(frow)

    def drain(buf, sem):
        def drow(p):
            pltpu.make_async_copy(
                buf.at[pl.ds(0, NB)], out_hbm.at[0, pl.ds(0, NB)],
                sem).wait()
        pl.loop(0, P)(drow)

    def wait_idx():
        def wrow(p):
            pltpu.make_async_copy(
                idx_hbm.at[0, pl.ds(0, BATCH_PER_W)],
                idx_v.at[0, pl.ds(0, BATCH_PER_W)], sem_i).wait()
        pl.loop(0, P)(wrow)

    zero_buf(rows0)
    wait_idx()
    scatter(0, rows0, ones)
    fire(0, rows0, sem0)
    for b in range(1, NBUF):
        zero_buf(bufs[b])
        scatter(b, bufs[b], ones)
        fire(b, bufs[b], sems[b])

    def step(c):
        for b in range(NBUF):
            chunk = c + b
            drain(bufs[b], sems[b])
            scatter(chunk - NBUF, bufs[b], zeros)
            scatter(chunk, bufs[b], ones)
            fire(chunk, bufs[b], sems[b])

    pl.loop(NBUF, CHUNKS, step=NBUF)(step)

    for b in range(NBUF):
        drain(bufs[b], sems[b])


@functools.partial(jax.jit, static_argnames=())
def kernel(index_feature):
    idx_t = jnp.transpose(index_feature).astype(jnp.int32)
    sc_kernel = pl.kernel(
        _body,
        out_type=jax.ShapeDtypeStruct((P, B, C), jnp.float32),
        mesh=plsc.VectorSubcoreMesh(
            core_axis_name="c", subcore_axis_name="s",
            num_cores=NC, num_subcores=NS),
        compiler_params=pltpu.CompilerParams(needs_layout_passes=False),
        scratch_types=[
            pltpu.VMEM((P, IDXCOL), jnp.int32),
            pltpu.VMEM((RPC, C), jnp.float32),
            pltpu.VMEM((RPC, C), jnp.float32),
            pltpu.VMEM((RPC, C), jnp.float32),
            pltpu.VMEM((RPC, C), jnp.float32),
            pltpu.SemaphoreType.DMA,
            pltpu.SemaphoreType.DMA,
            pltpu.SemaphoreType.DMA,
            pltpu.SemaphoreType.DMA,
            pltpu.SemaphoreType.DMA,
        ],
    )
    out_pm = sc_kernel(idx_t)
    return jnp.transpose(out_pm, (1, 0, 2))


# speedup vs baseline: 1.0252x; 1.0252x over previous
import functools

import jax
import jax.numpy as jnp
from jax import lax
from jax.experimental import pallas as pl
from jax.experimental.pallas import tpu as pltpu
from jax.experimental.pallas import tpu_sc as plsc

B, P, C = 16384, 26, 128
NC, NS, L = 2, 16, 16
NW = NC * NS
BATCH_PER_W = B // NW
NB = 16
RPC = NB * P
CHUNKS = BATCH_PER_W // NB


def _body(idx_hbm, out_hbm, idx_v, rows0, rows1, sem_i, sem0, sem1):
    wid = lax.axis_index("s") * NC + lax.axis_index("c")
    base_b = wid * BATCH_PER_W

    def stage_idx(p):
        pltpu.async_copy(
            idx_hbm.at[p, pl.ds(base_b, BATCH_PER_W)], idx_v.at[p], sem_i)

    pl.loop(0, P)(stage_idx)

    zeros = jnp.zeros((L,), jnp.float32)
    ones = jnp.ones((L,), jnp.float32)
    lanes = lax.iota(jnp.int32, L)

    bufs = (rows0, rows1)
    sems = (sem0, sem1)

    def zero_buf(buf):
        def zrow(i):
            for k in range(C // L):
                buf[i, pl.ds(k * L, L)] = zeros
        pl.loop(0, RPC)(zrow)

    def scatter(chunk, buf, val):
        def srow(p):
            iv = idx_v[p, pl.ds(chunk * NB, NB)]
            plsc.store_scatter(buf, [lanes + p * NB, iv], val)
        pl.loop(0, P)(srow)

    def fire(chunk, buf, sem):
        def frow(p):
            dst = out_hbm.at[p, pl.ds(base_b + chunk * NB, NB)]
            pltpu.async_copy(buf.at[pl.ds(p * NB, NB)], dst, sem)
        pl.loop(0, P)(frow)

    def drain(buf, sem):
        def drow(p):
            pltpu.make_async_copy(
                buf.at[pl.ds(0, NB)], out_hbm.at[0, pl.ds(0, NB)],
                sem).wait()
        pl.loop(0, P)(drow)

    def wait_idx():
        def wrow(p):
            pltpu.make_async_copy(
                idx_hbm.at[0, pl.ds(0, BATCH_PER_W)], idx_v.at[0],
                sem_i).wait()
        pl.loop(0, P)(wrow)

    zero_buf(rows0)
    wait_idx()
    scatter(0, rows0, ones)
    fire(0, rows0, sem0)
    zero_buf(rows1)
    scatter(1, rows1, ones)
    fire(1, rows1, sem1)

    def step(c):
        for b in range(2):
            chunk = c + b
            drain(bufs[b], sems[b])
            scatter(chunk - 2, bufs[b], zeros)
            scatter(chunk, bufs[b], ones)
            fire(chunk, bufs[b], sems[b])

    pl.loop(2, CHUNKS, step=2)(step)

    for b in range(2):
        drain(bufs[b], sems[b])


@functools.partial(jax.jit, static_argnames=())
def kernel(index_feature):
    idx_t = jnp.transpose(index_feature).astype(jnp.int32)
    sc_kernel = pl.kernel(
        _body,
        out_type=jax.ShapeDtypeStruct((P, B, C), jnp.float32),
        mesh=plsc.VectorSubcoreMesh(
            core_axis_name="c", subcore_axis_name="s",
            num_cores=NC, num_subcores=NS),
        compiler_params=pltpu.CompilerParams(needs_layout_passes=False),
        scratch_types=[
            pltpu.VMEM((P, BATCH_PER_W), jnp.int32),
            pltpu.VMEM((RPC, C), jnp.float32),
            pltpu.VMEM((RPC, C), jnp.float32),
            pltpu.SemaphoreType.DMA,
            pltpu.SemaphoreType.DMA,
            pltpu.SemaphoreType.DMA,
        ],
    )
    out_pm = sc_kernel(idx_t)
    return jnp.transpose(out_pm, (1, 0, 2))


# speedup vs baseline: 1.0523x; 1.0264x over previous
import functools

import jax
import jax.numpy as jnp
from jax import lax
from jax.experimental import pallas as pl
from jax.experimental.pallas import tpu as pltpu
from jax.experimental.pallas import tpu_sc as plsc

B, P, C = 16384, 26, 128
NC, NS, L = 2, 16, 16
NW = NC * NS
ROWS_PER_W = P * B // NW
NR = 256
CHUNKS = ROWS_PER_W // NR
CPB = B // NR


def _body(idx_hbm, out_hbm, idx_v, rows0, rows1, sem_i, sem0, sem1):
    wid = lax.axis_index("s") * NC + lax.axis_index("c")
    g0 = wid * CHUNKS
    p0 = g0 // CPB
    p1 = jnp.minimum(p0 + 1, P - 1)

    pltpu.async_copy(idx_hbm.at[p0], idx_v.at[0], sem_i)
    pltpu.async_copy(idx_hbm.at[p1], idx_v.at[1], sem_i)

    zeros = jnp.zeros((L,), jnp.float32)
    ones = jnp.ones((L,), jnp.float32)
    lanes = lax.iota(jnp.int32, L)

    bufs = (rows0, rows1)
    sems = (sem0, sem1)

    def zero_buf(buf):
        def zrow(i):
            for k in range(C // L):
                buf[i, pl.ds(k * L, L)] = zeros
        pl.loop(0, NR)(zrow)

    def scatter(g, buf, val):
        p = g // CPB
        b0 = (g - p * CPB) * NR

        def srow(j):
            iv = idx_v[p - p0, pl.ds(b0 + j * L, L)]
            plsc.store_scatter(buf, [lanes + j * L, iv], val)
        pl.loop(0, NR // L)(srow)

    def fire(g, buf, sem):
        p = g // CPB
        b0 = (g - p * CPB) * NR
        pltpu.async_copy(buf, out_hbm.at[p, pl.ds(b0, NR)], sem)

    def drain(buf, sem):
        pltpu.make_async_copy(
            buf, out_hbm.at[0, pl.ds(0, NR)], sem).wait()

    def wait_idx():
        for _ in range(2):
            pltpu.make_async_copy(
                idx_hbm.at[0], idx_v.at[0], sem_i).wait()

    zero_buf(rows0)
    wait_idx()
    scatter(g0, rows0, ones)
    fire(g0, rows0, sem0)
    zero_buf(rows1)
    scatter(g0 + 1, rows1, ones)
    fire(g0 + 1, rows1, sem1)

    def step(c):
        for b in range(2):
            g = g0 + c + b
            drain(bufs[b], sems[b])
            scatter(g - 2, bufs[b], zeros)
            scatter(g, bufs[b], ones)
            fire(g, bufs[b], sems[b])

    pl.loop(2, CHUNKS, step=2)(step)

    for b in range(2):
        drain(bufs[b], sems[b])


@functools.partial(jax.jit, static_argnames=())
def kernel(index_feature):
    idx_t = jnp.transpose(index_feature).astype(jnp.int32)
    sc_kernel = pl.kernel(
        _body,
        out_type=jax.ShapeDtypeStruct((P, B, C), jnp.float32),
        mesh=plsc.VectorSubcoreMesh(
            core_axis_name="c", subcore_axis_name="s",
            num_cores=NC, num_subcores=NS),
        compiler_params=pltpu.CompilerParams(needs_layout_passes=False),
        scratch_types=[
            pltpu.VMEM((2, B), jnp.int32),
            pltpu.VMEM((NR, C), jnp.float32),
            pltpu.VMEM((NR, C), jnp.float32),
            pltpu.SemaphoreType.DMA,
            pltpu.SemaphoreType.DMA,
            pltpu.SemaphoreType.DMA,
        ],
    )
    out_pm = sc_kernel(idx_t)
    return jnp.transpose(out_pm, (1, 0, 2))
